# Initial kernel scaffold; baseline (speedup 1.0000x reference)
import jax
import jax.numpy as jnp
from jax.experimental import pallas as pl


def kernel(X, edge_index, edge_weight, params):
    raise NotImplementedError("write your pallas kernel here")


# speedup vs baseline: 8.2615x; 8.2615x over previous
import functools

import jax
import jax.numpy as jnp
from jax import lax
from jax.experimental import pallas as pl
from jax.experimental.pallas import tpu as pltpu
from jax.experimental.pallas import tpu_sc as plsc

N = 50000
E = 800000
T = 3
F_IN = 4
H = 64
HH = 32
NSUB = 16
CH = 128
SB = 16
EPS_SB = CH * SB
NSB = 25
E_PAD = NSUB * NSB * EPS_SB
NA = 50048
RPS = NA // NSUB
DRPS = NA // NSUB
ZR = 184
DSB = 8
DNSB = (E_PAD // CH) // (2 * NSUB * DSB)
BN_EPS = 1e-5
BLK = 2000
NB = N // BLK
_F32 = jnp.float32


def _mesh():
    return plsc.VectorSubcoreMesh(
        core_axis_name="c", subcore_axis_name="s", num_cores=2, num_subcores=16
    )


def _deg_body(dst2, ew2, d0, d1, acc1, zb1, dstbuf, ewbuf):
    c = lax.axis_index("c")
    s = lax.axis_index("s")

    def run(out):
        def zrow(i, _):
            zb1[pl.ds(i * 16, 16)] = jnp.zeros((16,), _F32)
            return 0

        lax.fori_loop(0, DRPS // 16, zrow, 0)
        zb1[pl.ds(DRPS - 16, 16)] = jnp.zeros((16,), _F32)
        pltpu.sync_copy(zb1, acc1.at[pl.ds(s * DRPS, DRPS)])
        plsc.subcore_barrier()
        wid_local = s
        half = c * ((E_PAD // CH) // 2)

        def per_sb(sb, _):
            row0 = half + (wid_local * DNSB + sb) * DSB
            pltpu.sync_copy(dst2.at[pl.ds(row0, DSB)], dstbuf)
            pltpu.sync_copy(ew2.at[pl.ds(row0, DSB)], ewbuf)
            for j in range(DSB):
                pltpu.sync_copy(ewbuf.at[j], acc1.at[dstbuf.at[j]], add=True)
            return 0

        lax.fori_loop(0, DNSB, per_sb, 0)
        plsc.subcore_barrier()
        pltpu.sync_copy(acc1.at[pl.ds(s * DRPS, DRPS)], zb1)
        pltpu.sync_copy(zb1, out.at[pl.ds(s * DRPS, DRPS)])

    @pl.when(c == 0)
    def _():
        run(d0)

    @pl.when(c == 1)
    def _():
        run(d1)


def _sc_deg(dst2, ew2):
    f = pl.kernel(
        _deg_body,
        out_type=[jax.ShapeDtypeStruct((NA,), _F32)] * 2,
        mesh=_mesh(),
        scratch_types=[
            pltpu.VMEM_SHARED((NA,), _F32),
            pltpu.VMEM((DRPS,), _F32),
            pltpu.VMEM((DSB, CH), jnp.int32),
            pltpu.VMEM((DSB, CH), _F32),
        ],
    )
    return f(dst2, ew2)


_GDN = lax.GatherDimensionNumbers(
    offset_dims=(), collapsed_slice_dims=(0,), start_index_map=(0,)
)


def _splat(wvec, m):
    idx = jnp.full((16, 1), m, jnp.int32)
    return lax.gather(wvec, idx, dimension_numbers=_GDN, slice_sizes=(1,),
                      mode=lax.GatherScatterMode.PROMISE_IN_BOUNDS)


def _spmm_body(src3, dst2, ew2, ys0, ys1, out0, out1,
               acc, zbuf, idxbuf, dstbuf, ewbuf, rows_a, rows_b, sem_a, sem_b):
    c = lax.axis_index("c")
    s = lax.axis_index("s")

    def scan_edges(t, ys):
        def per_sb(sb, _):
            row0 = (s * NSB + sb) * SB
            pltpu.sync_copy(src3.at[pl.ds(t * (E_PAD // CH) + row0, SB)], idxbuf)
            pltpu.sync_copy(dst2.at[pl.ds(row0, SB)], dstbuf)
            pltpu.sync_copy(ew2.at[pl.ds(row0, SB)], ewbuf)
            pend = {}
            pend[0] = pltpu.async_copy(ys.at[idxbuf.at[0]], rows_a, sem_a)
            pend[1] = pltpu.async_copy(ys.at[idxbuf.at[1]], rows_b, sem_b)
            for j in range(SB):
                rows, sem = (rows_a, sem_a) if j % 2 == 0 else (rows_b, sem_b)
                pend[j].wait()

                def scale_group(g, _):
                    wvec = ewbuf[j, pl.ds(g * 16, 16)]
                    for m in range(16):
                        w = _splat(wvec, m)
                        r = g * 16 + m
                        rows[r, pl.ds(0, 16)] = rows[r, pl.ds(0, 16)] * w
                        rows[r, pl.ds(16, 16)] = rows[r, pl.ds(16, 16)] * w
                    return 0

                lax.fori_loop(0, CH // 16, scale_group, 0)
                pltpu.sync_copy(rows, acc.at[dstbuf.at[j]], add=True)
                if j + 2 < SB:
                    pend[j + 2] = pltpu.async_copy(ys.at[idxbuf.at[j + 2]], rows, sem)
            return 0

        lax.fori_loop(0, NSB, per_sb, 0)

    def run_t(t, ys, out):
        def zrow(i, _):
            zbuf[i, pl.ds(0, 16)] = jnp.zeros((16,), _F32)
            zbuf[i, pl.ds(16, 16)] = jnp.zeros((16,), _F32)
            return 0

        lax.fori_loop(0, ZR, zrow, 0)

        def zcopy(i, _):
            pltpu.sync_copy(zbuf, acc.at[pl.ds(s * RPS + i * ZR, ZR)])
            return 0

        lax.fori_loop(0, RPS // ZR, zcopy, 0)
        plsc.subcore_barrier()
        scan_edges(t, ys)
        plsc.subcore_barrier()
        pltpu.sync_copy(acc.at[pl.ds(s * RPS, RPS)],
                        out.at[pl.ds(t * NA + s * RPS, RPS)])

    def per_t(t, _):
        @pl.when(c == 0)
        def _():
            run_t(t, ys0, out0)

        @pl.when(c == 1)
        def _():
            run_t(t, ys1, out1)

        return 0

    lax.fori_loop(0, T, per_t, 0)


def _sc_spmm(src3, dst2, ew2, ys0, ys1):
    f = pl.kernel(
        _spmm_body,
        out_type=[jax.ShapeDtypeStruct((T * NA, HH), _F32)] * 2,
        mesh=_mesh(),
        compiler_params=pltpu.CompilerParams(use_tc_tiling_on_sc=False),
        scratch_types=[
            pltpu.VMEM_SHARED((NA, HH), _F32),
            pltpu.VMEM((ZR, HH), _F32),
            pltpu.VMEM((SB, CH), jnp.int32),
            pltpu.VMEM((SB, CH), jnp.int32),
            pltpu.VMEM((SB, CH), _F32),
            pltpu.VMEM((CH, HH), _F32),
            pltpu.VMEM((CH, HH), _F32),
            pltpu.SemaphoreType.DMA,
            pltpu.SemaphoreType.DMA,
        ],
    )
    return f(src3, dst2, ew2, ys0, ys1)


def _a0_body(x_ref, da_ref, db_ref, w_ref, ys0_ref, ys1_ref, u_ref):
    u = lax.rsqrt(da_ref[:] + db_ref[:] + 1.0)
    y = jnp.dot(x_ref[0], w_ref[:], preferred_element_type=_F32)
    ys = y * u
    ys0_ref[:] = ys[:, :HH]
    ys1_ref[:] = ys[:, HH:]
    u_ref[:] = u


def _tc_a0(X, da, db, w0):
    return pl.pallas_call(
        _a0_body,
        grid=(T, NB),
        in_specs=[
            pl.BlockSpec((1, BLK, F_IN), lambda t, nb: (t, nb, 0)),
            pl.BlockSpec((BLK, 1), lambda t, nb: (nb, 0)),
            pl.BlockSpec((BLK, 1), lambda t, nb: (nb, 0)),
            pl.BlockSpec((F_IN, H), lambda t, nb: (0, 0)),
        ],
        out_specs=[
            pl.BlockSpec((BLK, HH), lambda t, nb: (t * NB + nb, 0)),
            pl.BlockSpec((BLK, HH), lambda t, nb: (t * NB + nb, 0)),
            pl.BlockSpec((BLK, 1), lambda t, nb: (nb, 0)),
        ],
        out_shape=[
            jax.ShapeDtypeStruct((T * N, HH), _F32),
            jax.ShapeDtypeStruct((T * N, HH), _F32),
            jax.ShapeDtypeStruct((N, 1), _F32),
        ],
    )(X, da, db, w0)


def _ai_body(r_ref, st_ref, g_ref, be_ref, w_ref, u_ref, ys0_ref, ys1_ref):
    s1 = st_ref[0, 0]
    s2 = st_ref[0, 1]
    m = s1 / N
    v = s2 / N - m * m
    sc = g_ref[:] * lax.rsqrt(v + BN_EPS)
    h = (r_ref[:] - m[None, :]) * sc[None, :] + be_ref[:][None, :]
    y = jnp.dot(h, w_ref[:], preferred_element_type=_F32)
    ys = y * u_ref[:]
    ys0_ref[:] = ys[:, :HH]
    ys1_ref[:] = ys[:, HH:]


def _tc_ai(r, st, g, be, w, u):
    return pl.pallas_call(
        _ai_body,
        grid=(T, NB),
        in_specs=[
            pl.BlockSpec((BLK, H), lambda t, nb: (t * NB + nb, 0)),
            pl.BlockSpec((1, 8, H), lambda t, nb: (t, 0, 0)),
            pl.BlockSpec((H,), lambda t, nb: (0,)),
            pl.BlockSpec((H,), lambda t, nb: (0,)),
            pl.BlockSpec((H, H), lambda t, nb: (0, 0)),
            pl.BlockSpec((BLK, 1), lambda t, nb: (nb, 0)),
        ],
        out_specs=[
            pl.BlockSpec((BLK, HH), lambda t, nb: (t * NB + nb, 0)),
            pl.BlockSpec((BLK, HH), lambda t, nb: (t * NB + nb, 0)),
        ],
        out_shape=[
            jax.ShapeDtypeStruct((T * N, HH), _F32),
            jax.ShapeDtypeStruct((T * N, HH), _F32),
        ],
    )(r, st, g, be, w, u)


def _b_body(p0_ref, p1_ref, ys0_ref, ys1_ref, u_ref, bc_ref, r_ref, st_ref):
    nb = pl.program_id(1)
    p = jnp.concatenate([p0_ref[0], p1_ref[0]], axis=1)
    ysf = jnp.concatenate([ys0_ref[:], ys1_ref[:]], axis=1)
    pre = u_ref[:] * (p + ysf) + bc_ref[:][None, :]
    r = jnp.maximum(pre, 0.0)
    r_ref[:] = r
    s1 = jnp.sum(r, axis=0)
    s2 = jnp.sum(r * r, axis=0)

    @pl.when(nb == 0)
    def _():
        st_ref[0, 0] = s1
        st_ref[0, 1] = s2

    @pl.when(nb != 0)
    def _():
        st_ref[0, 0] += s1
        st_ref[0, 1] += s2


def _tc_b(p0, p1, ys0, ys1, u, bc):
    return pl.pallas_call(
        _b_body,
        grid=(T, NB),
        in_specs=[
            pl.BlockSpec((1, BLK, HH), lambda t, nb: (t, nb, 0)),
            pl.BlockSpec((1, BLK, HH), lambda t, nb: (t, nb, 0)),
            pl.BlockSpec((BLK, HH), lambda t, nb: (t * NB + nb, 0)),
            pl.BlockSpec((BLK, HH), lambda t, nb: (t * NB + nb, 0)),
            pl.BlockSpec((BLK, 1), lambda t, nb: (nb, 0)),
            pl.BlockSpec((H,), lambda t, nb: (0,)),
        ],
        out_specs=[
            pl.BlockSpec((BLK, H), lambda t, nb: (t * NB + nb, 0)),
            pl.BlockSpec((1, 8, H), lambda t, nb: (t, 0, 0)),
        ],
        out_shape=[
            jax.ShapeDtypeStruct((T * N, H), _F32),
            jax.ShapeDtypeStruct((T, 8, H), _F32),
        ],
    )(p0, p1, ys0, ys1, u, bc)


def _lstm_body(r0_ref, r1_ref, r2_ref, st_ref, g_ref, be_ref, s_ref,
               wih0_ref, whh0_ref, bi0_ref, bh0_ref,
               wih1_ref, whh1_ref, bi1_ref, bh1_ref,
               wl1h_ref, wl1s_ref, bl1_ref, wl2_ref, bl2_ref, out_ref):
    b0 = (bi0_ref[:] + bh0_ref[:])[None, :]
    b1 = (bi1_ref[:] + bh1_ref[:])[None, :]
    h0 = jnp.zeros((BLK, H), _F32)
    c0 = jnp.zeros((BLK, H), _F32)
    h1 = jnp.zeros((BLK, H), _F32)
    c1 = jnp.zeros((BLK, H), _F32)
    rrefs = (r0_ref, r1_ref, r2_ref)
    for t in range(T):
        s1 = st_ref[t, 0]
        s2 = st_ref[t, 1]
        m = s1 / N
        v = s2 / N - m * m
        sc = g_ref[:] * lax.rsqrt(v + BN_EPS)
        x = (rrefs[t][:] - m[None, :]) * sc[None, :] + be_ref[:][None, :]
        gt = (jnp.dot(x, wih0_ref[:], preferred_element_type=_F32)
              + jnp.dot(h0, whh0_ref[:], preferred_element_type=_F32) + b0)
        ig = jax.nn.sigmoid(gt[:, 0:H])
        fg = jax.nn.sigmoid(gt[:, H:2 * H])
        gg = jnp.tanh(gt[:, 2 * H:3 * H])
        og = jax.nn.sigmoid(gt[:, 3 * H:4 * H])
        c0 = fg * c0 + ig * gg
        h0 = og * jnp.tanh(c0)
        gt = (jnp.dot(h0, wih1_ref[:], preferred_element_type=_F32)
              + jnp.dot(h1, whh1_ref[:], preferred_element_type=_F32) + b1)
        ig = jax.nn.sigmoid(gt[:, 0:H])
        fg = jax.nn.sigmoid(gt[:, H:2 * H])
        gg = jnp.tanh(gt[:, 2 * H:3 * H])
        og = jax.nn.sigmoid(gt[:, 3 * H:4 * H])
        c1 = fg * c1 + ig * gg
        h1 = og * jnp.tanh(c1)
    hf = jnp.maximum(h1, 0.0)
    o = (jnp.dot(hf, wl1h_ref[:], preferred_element_type=_F32)
         + jnp.dot(s_ref[:], wl1s_ref[:], preferred_element_type=_F32)
         + bl1_ref[:][None, :])
    out_ref[:] = jnp.sum(o * wl2_ref[:], axis=1, keepdims=True) + bl2_ref[0, 0]


def _tc_lstm(r, st, g, be, s, wih0, whh0, bi0, bh0, wih1, whh1, bi1, bh1,
             wl1h, wl1s, bl1, wl2, bl2):
    vec = lambda d: pl.BlockSpec((d,), lambda nb: (0,))
    mat = lambda a, b: pl.BlockSpec((a, b), lambda nb: (0, 0))
    return pl.pallas_call(
        _lstm_body,
        grid=(NB,),
        in_specs=[
            pl.BlockSpec((BLK, H), lambda nb: (nb, 0)),
            pl.BlockSpec((BLK, H), lambda nb: (NB + nb, 0)),
            pl.BlockSpec((BLK, H), lambda nb: (2 * NB + nb, 0)),
            pl.BlockSpec((T, 8, H), lambda nb: (0, 0, 0)),
            vec(H), vec(H),
            pl.BlockSpec((BLK, 8), lambda nb: (nb, 0)),
            mat(H, 4 * H), mat(H, 4 * H), vec(4 * H), vec(4 * H),
            mat(H, 4 * H), mat(H, 4 * H), vec(4 * H), vec(4 * H),
            mat(H, H), mat(8, H), vec(H), mat(1, H), mat(1, 1),
        ],
        out_specs=pl.BlockSpec((BLK, 1), lambda nb: (nb, 0)),
        out_shape=jax.ShapeDtypeStruct((N, 1), _F32),
    )(r, r, r, st, g, be, s, wih0, whh0, bi0, bh0, wih1, whh1, bi1, bh1,
      wl1h, wl1s, bl1, wl2, bl2)


def kernel(X, edge_index, edge_weight, params):
    p = params
    src = edge_index[0]
    dst = edge_index[1]
    pad = E_PAD - E
    src_p = jnp.concatenate([src, jnp.zeros((pad,), jnp.int32)])
    dst_p = jnp.concatenate([dst, jnp.zeros((pad,), jnp.int32)])
    ew_p = jnp.concatenate([edge_weight, jnp.zeros((pad,), _F32)])
    toff = (jnp.arange(T, dtype=jnp.int32) * N)[:, None]
    src3 = (src_p[None, :] + toff).reshape(T * E_PAD // CH, CH)
    dst2 = dst_p.reshape(E_PAD // CH, CH)
    ew2 = ew_p.reshape(E_PAD // CH, CH)

    da, db = _sc_deg(dst2, ew2)
    da = da[:N].reshape(N, 1)
    db = db[:N].reshape(N, 1)

    ys0, ys1, u = _tc_a0(X, da, db, p["W0"])
    for i in range(4):
        p0, p1 = _sc_spmm(src3, dst2, ew2, ys0, ys1)
        p0 = p0.reshape(T, NA, HH)
        p1 = p1.reshape(T, NA, HH)
        r, st = _tc_b(p0, p1, ys0, ys1, u, p[f"bc{i}"])
        if i < 3:
            ys0, ys1 = _tc_ai(r, st, p[f"g{i}"], p[f"be{i}"], p[f"W{i + 1}"], u)

    s_mat = jnp.pad(X[:, :, 0].T, ((0, 0), (0, 5)))
    wl1h = p["Wl1"][:H]
    wl1s = jnp.pad(p["Wl1"][H:], ((0, 5), (0, 0)))
    out = _tc_lstm(
        r, st, p["g3"], p["be3"], s_mat,
        p["Wih0"].T, p["Whh0"].T, p["bih0"], p["bhh0"],
        p["Wih1"].T, p["Whh1"].T, p["bih1"], p["bhh1"],
        wl1h, wl1s, p["bl1"], p["Wl2"].T, p["bl2"].reshape(1, 1),
    )
    return out


# speedup vs baseline: 8.9322x; 1.0812x over previous
import functools

import jax
import jax.numpy as jnp
from jax import lax
from jax.experimental import pallas as pl
from jax.experimental.pallas import tpu as pltpu
from jax.experimental.pallas import tpu_sc as plsc

N = 50000
E = 800000
T = 3
F_IN = 4
H = 64
HH = 32
NSUB = 16
CH = 128
SB = 16
EPS_SB = CH * SB
NSB = 25
E_PAD = NSUB * NSB * EPS_SB
NA = 50048
RPS = NA // NSUB
DRPS = NA // NSUB
ZR = 136
DSB = 8
DNSB = (E_PAD // CH) // (2 * NSUB * DSB)
BN_EPS = 1e-5
BLK = 2000
NB = N // BLK
_F32 = jnp.float32


def _mesh():
    return plsc.VectorSubcoreMesh(
        core_axis_name="c", subcore_axis_name="s", num_cores=2, num_subcores=16
    )


def _deg_body(dst2, ew2, d0, d1, acc1, zb1, dstbuf, ewbuf):
    c = lax.axis_index("c")
    s = lax.axis_index("s")

    def run(out):
        def zrow(i, _):
            zb1[pl.ds(i * 16, 16)] = jnp.zeros((16,), _F32)
            return 0

        lax.fori_loop(0, DRPS // 16, zrow, 0)
        zb1[pl.ds(DRPS - 16, 16)] = jnp.zeros((16,), _F32)
        pltpu.sync_copy(zb1, acc1.at[pl.ds(s * DRPS, DRPS)])
        plsc.subcore_barrier()
        wid_local = s
        half = c * ((E_PAD // CH) // 2)

        def per_sb(sb, _):
            row0 = half + (wid_local * DNSB + sb) * DSB
            pltpu.sync_copy(dst2.at[pl.ds(row0, DSB)], dstbuf)
            pltpu.sync_copy(ew2.at[pl.ds(row0, DSB)], ewbuf)
            for j in range(DSB):
                pltpu.sync_copy(ewbuf.at[j], acc1.at[dstbuf.at[j]], add=True)
            return 0

        lax.fori_loop(0, DNSB, per_sb, 0)
        plsc.subcore_barrier()
        pltpu.sync_copy(acc1.at[pl.ds(s * DRPS, DRPS)], zb1)
        pltpu.sync_copy(zb1, out.at[pl.ds(s * DRPS, DRPS)])

    @pl.when(c == 0)
    def _():
        run(d0)

    @pl.when(c == 1)
    def _():
        run(d1)


def _sc_deg(dst2, ew2):
    f = pl.kernel(
        _deg_body,
        out_type=[jax.ShapeDtypeStruct((NA,), _F32)] * 2,
        mesh=_mesh(),
        scratch_types=[
            pltpu.VMEM_SHARED((NA,), _F32),
            pltpu.VMEM((DRPS,), _F32),
            pltpu.VMEM((DSB, CH), jnp.int32),
            pltpu.VMEM((DSB, CH), _F32),
        ],
    )
    return f(dst2, ew2)


_GDN = lax.GatherDimensionNumbers(
    offset_dims=(), collapsed_slice_dims=(0,), start_index_map=(0,)
)


def _splat(wvec, m):
    idx = jnp.full((16, 1), m, jnp.int32)
    return lax.gather(wvec, idx, dimension_numbers=_GDN, slice_sizes=(1,),
                      mode=lax.GatherScatterMode.PROMISE_IN_BOUNDS)


def _spmm_body(src3, dst2, ew2, ys0, ys1, out0, out1,
               acc, zbuf, idxbuf, dstbuf, ewbuf,
               rows0, rows1, rows2, rows3,
               gs0, gs1, gs2, gs3, ss0, ss1, ss2, ss3):
    c = lax.axis_index("c")
    s = lax.axis_index("s")
    bufs = (rows0, rows1, rows2, rows3)
    gsems = (gs0, gs1, gs2, gs3)
    ssems = (ss0, ss1, ss2, ss3)

    def scale(rows, j):
        def scale_group(g, _):
            wvec = ewbuf[j, pl.ds(g * 16, 16)]
            for m in range(16):
                w = _splat(wvec, m)
                r = g * 16 + m
                rows[r, pl.ds(0, 16)] = rows[r, pl.ds(0, 16)] * w
                rows[r, pl.ds(16, 16)] = rows[r, pl.ds(16, 16)] * w
            return 0

        lax.fori_loop(0, CH // 16, scale_group, 0)

    def scan_edges(t, ys):
        def per_sb(sb, _):
            row0 = (s * NSB + sb) * SB
            pltpu.sync_copy(src3.at[pl.ds(t * (E_PAD // CH) + row0, SB)], idxbuf)
            pltpu.sync_copy(dst2.at[pl.ds(row0, SB)], dstbuf)
            pltpu.sync_copy(ew2.at[pl.ds(row0, SB)], ewbuf)
            gp = {}
            sp = {}
            for j in range(3):
                gp[j] = pltpu.async_copy(ys.at[idxbuf.at[j]], bufs[j], gsems[j])
            for j in range(SB):
                b = j % 4
                gp[j].wait()
                scale(bufs[b], j)
                if j >= 1:
                    sp[j - 1].wait()
                sp[j] = pltpu.async_copy(
                    bufs[b], acc.at[dstbuf.at[j]], ssems[b], add=True)
                nxt = j + 3
                if nxt < SB:
                    bn = nxt % 4
                    gp[nxt] = pltpu.async_copy(
                        ys.at[idxbuf.at[nxt]], bufs[bn], gsems[bn])
            sp[SB - 1].wait()
            return 0

        lax.fori_loop(0, NSB, per_sb, 0)

    def run_t(t, ys, out):
        def zrow(i, _):
            zbuf[i, pl.ds(0, 16)] = jnp.zeros((16,), _F32)
            zbuf[i, pl.ds(16, 16)] = jnp.zeros((16,), _F32)
            return 0

        lax.fori_loop(0, ZR, zrow, 0)

        def zcopy(i, _):
            pltpu.sync_copy(zbuf, acc.at[pl.ds(s * RPS + i * ZR, ZR)])
            return 0

        lax.fori_loop(0, RPS // ZR, zcopy, 0)
        plsc.subcore_barrier()
        scan_edges(t, ys)
        plsc.subcore_barrier()
        pltpu.sync_copy(acc.at[pl.ds(s * RPS, RPS)],
                        out.at[pl.ds(t * NA + s * RPS, RPS)])

    def per_t(t, _):
        @pl.when(c == 0)
        def _():
            run_t(t, ys0, out0)

        @pl.when(c == 1)
        def _():
            run_t(t, ys1, out1)

        return 0

    lax.fori_loop(0, T, per_t, 0)


def _sc_spmm(src3, dst2, ew2, ys0, ys1):
    f = pl.kernel(
        _spmm_body,
        out_type=[jax.ShapeDtypeStruct((T * NA, HH), _F32)] * 2,
        mesh=_mesh(),
        compiler_params=pltpu.CompilerParams(use_tc_tiling_on_sc=False),
        scratch_types=[
            pltpu.VMEM_SHARED((NA, HH), _F32),
            pltpu.VMEM((ZR, HH), _F32),
            pltpu.VMEM((SB, CH), jnp.int32),
            pltpu.VMEM((SB, CH), jnp.int32),
            pltpu.VMEM((SB, CH), _F32),
            pltpu.VMEM((CH, HH), _F32),
            pltpu.VMEM((CH, HH), _F32),
            pltpu.VMEM((CH, HH), _F32),
            pltpu.VMEM((CH, HH), _F32),
        ] + [pltpu.SemaphoreType.DMA] * 8,
    )
    return f(src3, dst2, ew2, ys0, ys1)


def _a0_body(x_ref, da_ref, db_ref, w_ref, ys0_ref, ys1_ref, u_ref):
    u = lax.rsqrt(da_ref[:] + db_ref[:] + 1.0)
    y = jnp.dot(x_ref[0], w_ref[:], preferred_element_type=_F32)
    ys = y * u
    ys0_ref[:] = ys[:, :HH]
    ys1_ref[:] = ys[:, HH:]
    u_ref[:] = u


def _tc_a0(X, da, db, w0):
    return pl.pallas_call(
        _a0_body,
        grid=(T, NB),
        in_specs=[
            pl.BlockSpec((1, BLK, F_IN), lambda t, nb: (t, nb, 0)),
            pl.BlockSpec((BLK, 1), lambda t, nb: (nb, 0)),
            pl.BlockSpec((BLK, 1), lambda t, nb: (nb, 0)),
            pl.BlockSpec((F_IN, H), lambda t, nb: (0, 0)),
        ],
        out_specs=[
            pl.BlockSpec((BLK, HH), lambda t, nb: (t * NB + nb, 0)),
            pl.BlockSpec((BLK, HH), lambda t, nb: (t * NB + nb, 0)),
            pl.BlockSpec((BLK, 1), lambda t, nb: (nb, 0)),
        ],
        out_shape=[
            jax.ShapeDtypeStruct((T * N, HH), _F32),
            jax.ShapeDtypeStruct((T * N, HH), _F32),
            jax.ShapeDtypeStruct((N, 1), _F32),
        ],
    )(X, da, db, w0)


def _ai_body(r_ref, st_ref, g_ref, be_ref, w_ref, u_ref, ys0_ref, ys1_ref):
    s1 = st_ref[0, 0]
    s2 = st_ref[0, 1]
    m = s1 / N
    v = s2 / N - m * m
    sc = g_ref[:] * lax.rsqrt(v + BN_EPS)
    h = (r_ref[:] - m[None, :]) * sc[None, :] + be_ref[:][None, :]
    y = jnp.dot(h, w_ref[:], preferred_element_type=_F32)
    ys = y * u_ref[:]
    ys0_ref[:] = ys[:, :HH]
    ys1_ref[:] = ys[:, HH:]


def _tc_ai(r, st, g, be, w, u):
    return pl.pallas_call(
        _ai_body,
        grid=(T, NB),
        in_specs=[
            pl.BlockSpec((BLK, H), lambda t, nb: (t * NB + nb, 0)),
            pl.BlockSpec((1, 8, H), lambda t, nb: (t, 0, 0)),
            pl.BlockSpec((H,), lambda t, nb: (0,)),
            pl.BlockSpec((H,), lambda t, nb: (0,)),
            pl.BlockSpec((H, H), lambda t, nb: (0, 0)),
            pl.BlockSpec((BLK, 1), lambda t, nb: (nb, 0)),
        ],
        out_specs=[
            pl.BlockSpec((BLK, HH), lambda t, nb: (t * NB + nb, 0)),
            pl.BlockSpec((BLK, HH), lambda t, nb: (t * NB + nb, 0)),
        ],
        out_shape=[
            jax.ShapeDtypeStruct((T * N, HH), _F32),
            jax.ShapeDtypeStruct((T * N, HH), _F32),
        ],
    )(r, st, g, be, w, u)


def _b_body(p0_ref, p1_ref, ys0_ref, ys1_ref, u_ref, bc_ref, r_ref, st_ref):
    nb = pl.program_id(1)
    p = jnp.concatenate([p0_ref[0], p1_ref[0]], axis=1)
    ysf = jnp.concatenate([ys0_ref[:], ys1_ref[:]], axis=1)
    pre = u_ref[:] * (p + ysf) + bc_ref[:][None, :]
    r = jnp.maximum(pre, 0.0)
    r_ref[:] = r
    s1 = jnp.sum(r, axis=0)
    s2 = jnp.sum(r * r, axis=0)

    @pl.when(nb == 0)
    def _():
        st_ref[0, 0] = s1
        st_ref[0, 1] = s2

    @pl.when(nb != 0)
    def _():
        st_ref[0, 0] += s1
        st_ref[0, 1] += s2


def _tc_b(p0, p1, ys0, ys1, u, bc):
    return pl.pallas_call(
        _b_body,
        grid=(T, NB),
        in_specs=[
            pl.BlockSpec((1, BLK, HH), lambda t, nb: (t, nb, 0)),
            pl.BlockSpec((1, BLK, HH), lambda t, nb: (t, nb, 0)),
            pl.BlockSpec((BLK, HH), lambda t, nb: (t * NB + nb, 0)),
            pl.BlockSpec((BLK, HH), lambda t, nb: (t * NB + nb, 0)),
            pl.BlockSpec((BLK, 1), lambda t, nb: (nb, 0)),
            pl.BlockSpec((H,), lambda t, nb: (0,)),
        ],
        out_specs=[
            pl.BlockSpec((BLK, H), lambda t, nb: (t * NB + nb, 0)),
            pl.BlockSpec((1, 8, H), lambda t, nb: (t, 0, 0)),
        ],
        out_shape=[
            jax.ShapeDtypeStruct((T * N, H), _F32),
            jax.ShapeDtypeStruct((T, 8, H), _F32),
        ],
    )(p0, p1, ys0, ys1, u, bc)


def _lstm_body(r0_ref, r1_ref, r2_ref, st_ref, g_ref, be_ref, s_ref,
               wih0_ref, whh0_ref, bi0_ref, bh0_ref,
               wih1_ref, whh1_ref, bi1_ref, bh1_ref,
               wl1h_ref, wl1s_ref, bl1_ref, wl2_ref, bl2_ref, out_ref):
    b0 = (bi0_ref[:] + bh0_ref[:])[None, :]
    b1 = (bi1_ref[:] + bh1_ref[:])[None, :]
    h0 = jnp.zeros((BLK, H), _F32)
    c0 = jnp.zeros((BLK, H), _F32)
    h1 = jnp.zeros((BLK, H), _F32)
    c1 = jnp.zeros((BLK, H), _F32)
    rrefs = (r0_ref, r1_ref, r2_ref)
    for t in range(T):
        s1 = st_ref[t, 0]
        s2 = st_ref[t, 1]
        m = s1 / N
        v = s2 / N - m * m
        sc = g_ref[:] * lax.rsqrt(v + BN_EPS)
        x = (rrefs[t][:] - m[None, :]) * sc[None, :] + be_ref[:][None, :]
        gt = (jnp.dot(x, wih0_ref[:], preferred_element_type=_F32)
              + jnp.dot(h0, whh0_ref[:], preferred_element_type=_F32) + b0)
        ig = jax.nn.sigmoid(gt[:, 0:H])
        fg = jax.nn.sigmoid(gt[:, H:2 * H])
        gg = jnp.tanh(gt[:, 2 * H:3 * H])
        og = jax.nn.sigmoid(gt[:, 3 * H:4 * H])
        c0 = fg * c0 + ig * gg
        h0 = og * jnp.tanh(c0)
        gt = (jnp.dot(h0, wih1_ref[:], preferred_element_type=_F32)
              + jnp.dot(h1, whh1_ref[:], preferred_element_type=_F32) + b1)
        ig = jax.nn.sigmoid(gt[:, 0:H])
        fg = jax.nn.sigmoid(gt[:, H:2 * H])
        gg = jnp.tanh(gt[:, 2 * H:3 * H])
        og = jax.nn.sigmoid(gt[:, 3 * H:4 * H])
        c1 = fg * c1 + ig * gg
        h1 = og * jnp.tanh(c1)
    hf = jnp.maximum(h1, 0.0)
    o = (jnp.dot(hf, wl1h_ref[:], preferred_element_type=_F32)
         + jnp.dot(s_ref[:], wl1s_ref[:], preferred_element_type=_F32)
         + bl1_ref[:][None, :])
    out_ref[:] = jnp.sum(o * wl2_ref[:], axis=1, keepdims=True) + bl2_ref[0, 0]


def _tc_lstm(r, st, g, be, s, wih0, whh0, bi0, bh0, wih1, whh1, bi1, bh1,
             wl1h, wl1s, bl1, wl2, bl2):
    vec = lambda d: pl.BlockSpec((d,), lambda nb: (0,))
    mat = lambda a, b: pl.BlockSpec((a, b), lambda nb: (0, 0))
    return pl.pallas_call(
        _lstm_body,
        grid=(NB,),
        in_specs=[
            pl.BlockSpec((BLK, H), lambda nb: (nb, 0)),
            pl.BlockSpec((BLK, H), lambda nb: (NB + nb, 0)),
            pl.BlockSpec((BLK, H), lambda nb: (2 * NB + nb, 0)),
            pl.BlockSpec((T, 8, H), lambda nb: (0, 0, 0)),
            vec(H), vec(H),
            pl.BlockSpec((BLK, 8), lambda nb: (nb, 0)),
            mat(H, 4 * H), mat(H, 4 * H), vec(4 * H), vec(4 * H),
            mat(H, 4 * H), mat(H, 4 * H), vec(4 * H), vec(4 * H),
            mat(H, H), mat(8, H), vec(H), mat(1, H), mat(1, 1),
        ],
        out_specs=pl.BlockSpec((BLK, 1), lambda nb: (nb, 0)),
        out_shape=jax.ShapeDtypeStruct((N, 1), _F32),
    )(r, r, r, st, g, be, s, wih0, whh0, bi0, bh0, wih1, whh1, bi1, bh1,
      wl1h, wl1s, bl1, wl2, bl2)


def kernel(X, edge_index, edge_weight, params):
    p = params
    src = edge_index[0]
    dst = edge_index[1]
    pad = E_PAD - E
    src_p = jnp.concatenate([src, jnp.zeros((pad,), jnp.int32)])
    dst_p = jnp.concatenate([dst, jnp.zeros((pad,), jnp.int32)])
    ew_p = jnp.concatenate([edge_weight, jnp.zeros((pad,), _F32)])
    toff = (jnp.arange(T, dtype=jnp.int32) * N)[:, None]
    src3 = (src_p[None, :] + toff).reshape(T * E_PAD // CH, CH)
    dst2 = dst_p.reshape(E_PAD // CH, CH)
    ew2 = ew_p.reshape(E_PAD // CH, CH)

    da, db = _sc_deg(dst2, ew2)
    da = da[:N].reshape(N, 1)
    db = db[:N].reshape(N, 1)

    ys0, ys1, u = _tc_a0(X, da, db, p["W0"])
    for i in range(4):
        p0, p1 = _sc_spmm(src3, dst2, ew2, ys0, ys1)
        p0 = p0.reshape(T, NA, HH)
        p1 = p1.reshape(T, NA, HH)
        r, st = _tc_b(p0, p1, ys0, ys1, u, p[f"bc{i}"])
        if i < 3:
            ys0, ys1 = _tc_ai(r, st, p[f"g{i}"], p[f"be{i}"], p[f"W{i + 1}"], u)

    s_mat = jnp.pad(X[:, :, 0].T, ((0, 0), (0, 5)))
    wl1h = p["Wl1"][:H]
    wl1s = jnp.pad(p["Wl1"][H:], ((0, 5), (0, 0)))
    out = _tc_lstm(
        r, st, p["g3"], p["be3"], s_mat,
        p["Wih0"].T, p["Whh0"].T, p["bih0"], p["bhh0"],
        p["Wih1"].T, p["Whh1"].T, p["bih1"], p["bhh1"],
        wl1h, wl1s, p["bl1"], p["Wl2"].T, p["bl2"].reshape(1, 1),
    )
    return out


# speedup vs baseline: 9.2752x; 1.0384x over previous
import functools

import jax
import jax.numpy as jnp
from jax import lax
from jax.experimental import pallas as pl
from jax.experimental.pallas import tpu as pltpu
from jax.experimental.pallas import tpu_sc as plsc

N = 50000
E = 800000
T = 3
F_IN = 4
H = 64
HH = 32
NSUB = 16
CH = 128
SB = 16
EPS_SB = CH * SB
NSB = 25
E_PAD = NSUB * NSB * EPS_SB
NA = 50048
RPS = NA // NSUB
DRPS = NA // NSUB
ZR = 136
DSB = 8
DNSB = (E_PAD // CH) // (2 * NSUB * DSB)
BN_EPS = 1e-5
BLK = 2000
NB = N // BLK
_F32 = jnp.float32


def _mesh():
    return plsc.VectorSubcoreMesh(
        core_axis_name="c", subcore_axis_name="s", num_cores=2, num_subcores=16
    )


def _deg_body(dst2, ew2, d0, d1, acc1, zb1, dstbuf, ewbuf):
    c = lax.axis_index("c")
    s = lax.axis_index("s")

    def run(out):
        def zrow(i, _):
            zb1[pl.ds(i * 16, 16)] = jnp.zeros((16,), _F32)
            return 0

        lax.fori_loop(0, DRPS // 16, zrow, 0)
        zb1[pl.ds(DRPS - 16, 16)] = jnp.zeros((16,), _F32)
        pltpu.sync_copy(zb1, acc1.at[pl.ds(s * DRPS, DRPS)])
        plsc.subcore_barrier()
        wid_local = s
        half = c * ((E_PAD // CH) // 2)

        def per_sb(sb, _):
            row0 = half + (wid_local * DNSB + sb) * DSB
            pltpu.sync_copy(dst2.at[pl.ds(row0, DSB)], dstbuf)
            pltpu.sync_copy(ew2.at[pl.ds(row0, DSB)], ewbuf)
            for j in range(DSB):
                pltpu.sync_copy(ewbuf.at[j], acc1.at[dstbuf.at[j]], add=True)
            return 0

        lax.fori_loop(0, DNSB, per_sb, 0)
        plsc.subcore_barrier()
        pltpu.sync_copy(acc1.at[pl.ds(s * DRPS, DRPS)], zb1)
        pltpu.sync_copy(zb1, out.at[pl.ds(s * DRPS, DRPS)])

    @pl.when(c == 0)
    def _():
        run(d0)

    @pl.when(c == 1)
    def _():
        run(d1)


def _sc_deg(dst2, ew2):
    f = pl.kernel(
        _deg_body,
        out_type=[jax.ShapeDtypeStruct((NA,), _F32)] * 2,
        mesh=_mesh(),
        scratch_types=[
            pltpu.VMEM_SHARED((NA,), _F32),
            pltpu.VMEM((DRPS,), _F32),
            pltpu.VMEM((DSB, CH), jnp.int32),
            pltpu.VMEM((DSB, CH), _F32),
        ],
    )
    return f(dst2, ew2)


_GDN = lax.GatherDimensionNumbers(
    offset_dims=(), collapsed_slice_dims=(0,), start_index_map=(0,)
)


def _splat(wvec, m):
    idx = jnp.full((16, 1), m, jnp.int32)
    return lax.gather(wvec, idx, dimension_numbers=_GDN, slice_sizes=(1,),
                      mode=lax.GatherScatterMode.PROMISE_IN_BOUNDS)


def _spmm_body(src3, dst2, ew2, ys0, ys1, out0, out1,
               acc, zbuf, idxbuf, dstbuf, ewbuf,
               rows0, rows1, rows2, rows3,
               gs0, gs1, gs2, gs3, ss0, ss1, ss2, ss3, ms0, ms1, ms2, zs):
    c = lax.axis_index("c")
    s = lax.axis_index("s")
    bufs = (rows0, rows1, rows2, rows3)
    gsems = (gs0, gs1, gs2, gs3)
    ssems = (ss0, ss1, ss2, ss3)

    def scale(rows, j):
        def scale_group(g, _):
            wvec = ewbuf[j, pl.ds(g * 16, 16)]
            for m in range(16):
                w = _splat(wvec, m)
                r = g * 16 + m
                rows[r, pl.ds(0, 16)] = rows[r, pl.ds(0, 16)] * w
                rows[r, pl.ds(16, 16)] = rows[r, pl.ds(16, 16)] * w
            return 0

        lax.fori_loop(0, CH // 16, scale_group, 0)

    def scan_edges(t, ys):
        def per_sb(sb, _):
            row0 = (s * NSB + sb) * SB
            ma = pltpu.async_copy(
                src3.at[pl.ds(t * (E_PAD // CH) + row0, SB)], idxbuf, ms0)
            mb = pltpu.async_copy(dst2.at[pl.ds(row0, SB)], dstbuf, ms1)
            mc = pltpu.async_copy(ew2.at[pl.ds(row0, SB)], ewbuf, ms2)
            ma.wait()
            mb.wait()
            mc.wait()
            gp = {}
            sp = {}
            for j in range(3):
                gp[j] = pltpu.async_copy(ys.at[idxbuf.at[j]], bufs[j], gsems[j])
            for j in range(SB):
                b = j % 4
                gp[j].wait()
                scale(bufs[b], j)
                if j >= 1:
                    sp[j - 1].wait()
                sp[j] = pltpu.async_copy(
                    bufs[b], acc.at[dstbuf.at[j]], ssems[b], add=True)
                nxt = j + 3
                if nxt < SB:
                    bn = nxt % 4
                    gp[nxt] = pltpu.async_copy(
                        ys.at[idxbuf.at[nxt]], bufs[bn], gsems[bn])
            sp[SB - 1].wait()
            return 0

        lax.fori_loop(0, NSB, per_sb, 0)

    def run_t(t, ys, out):
        def zrow(i, _):
            zbuf[i, pl.ds(0, 16)] = jnp.zeros((16,), _F32)
            zbuf[i, pl.ds(16, 16)] = jnp.zeros((16,), _F32)
            return 0

        lax.fori_loop(0, ZR, zrow, 0)
        zp = [pltpu.async_copy(zbuf, acc.at[pl.ds(s * RPS + i * ZR, ZR)], zs)
              for i in range(RPS // ZR)]
        for d in zp:
            d.wait()
        plsc.subcore_barrier()
        scan_edges(t, ys)
        plsc.subcore_barrier()
        pltpu.sync_copy(acc.at[pl.ds(s * RPS, RPS)],
                        out.at[pl.ds(t * NA + s * RPS, RPS)])

    def per_t(t, _):
        @pl.when(c == 0)
        def _():
            run_t(t, ys0, out0)

        @pl.when(c == 1)
        def _():
            run_t(t, ys1, out1)

        return 0

    lax.fori_loop(0, T, per_t, 0)


def _sc_spmm(src3, dst2, ew2, ys0, ys1):
    f = pl.kernel(
        _spmm_body,
        out_type=[jax.ShapeDtypeStruct((T * NA, HH), _F32)] * 2,
        mesh=_mesh(),
        compiler_params=pltpu.CompilerParams(use_tc_tiling_on_sc=False),
        scratch_types=[
            pltpu.VMEM_SHARED((NA, HH), _F32),
            pltpu.VMEM((ZR, HH), _F32),
            pltpu.VMEM((SB, CH), jnp.int32),
            pltpu.VMEM((SB, CH), jnp.int32),
            pltpu.VMEM((SB, CH), _F32),
            pltpu.VMEM((CH, HH), _F32),
            pltpu.VMEM((CH, HH), _F32),
            pltpu.VMEM((CH, HH), _F32),
            pltpu.VMEM((CH, HH), _F32),
        ] + [pltpu.SemaphoreType.DMA] * 12,
    )
    return f(src3, dst2, ew2, ys0, ys1)


def _a0_body(x_ref, da_ref, db_ref, w_ref, ys0_ref, ys1_ref, u_ref):
    u = lax.rsqrt(da_ref[:] + db_ref[:] + 1.0)
    y = jnp.dot(x_ref[0], w_ref[:], preferred_element_type=_F32)
    ys = y * u
    ys0_ref[:] = ys[:, :HH]
    ys1_ref[:] = ys[:, HH:]
    u_ref[:] = u


def _tc_a0(X, da, db, w0):
    return pl.pallas_call(
        _a0_body,
        grid=(T, NB),
        in_specs=[
            pl.BlockSpec((1, BLK, F_IN), lambda t, nb: (t, nb, 0)),
            pl.BlockSpec((BLK, 1), lambda t, nb: (nb, 0)),
            pl.BlockSpec((BLK, 1), lambda t, nb: (nb, 0)),
            pl.BlockSpec((F_IN, H), lambda t, nb: (0, 0)),
        ],
        out_specs=[
            pl.BlockSpec((BLK, HH), lambda t, nb: (t * NB + nb, 0)),
            pl.BlockSpec((BLK, HH), lambda t, nb: (t * NB + nb, 0)),
            pl.BlockSpec((BLK, 1), lambda t, nb: (nb, 0)),
        ],
        out_shape=[
            jax.ShapeDtypeStruct((T * N, HH), _F32),
            jax.ShapeDtypeStruct((T * N, HH), _F32),
            jax.ShapeDtypeStruct((N, 1), _F32),
        ],
    )(X, da, db, w0)


def _ai_body(r_ref, st_ref, g_ref, be_ref, w_ref, u_ref, ys0_ref, ys1_ref):
    s1 = st_ref[0, 0]
    s2 = st_ref[0, 1]
    m = s1 / N
    v = s2 / N - m * m
    sc = g_ref[:] * lax.rsqrt(v + BN_EPS)
    h = (r_ref[:] - m[None, :]) * sc[None, :] + be_ref[:][None, :]
    y = jnp.dot(h, w_ref[:], preferred_element_type=_F32)
    ys = y * u_ref[:]
    ys0_ref[:] = ys[:, :HH]
    ys1_ref[:] = ys[:, HH:]


def _tc_ai(r, st, g, be, w, u):
    return pl.pallas_call(
        _ai_body,
        grid=(T, NB),
        in_specs=[
            pl.BlockSpec((BLK, H), lambda t, nb: (t * NB + nb, 0)),
            pl.BlockSpec((1, 8, H), lambda t, nb: (t, 0, 0)),
            pl.BlockSpec((H,), lambda t, nb: (0,)),
            pl.BlockSpec((H,), lambda t, nb: (0,)),
            pl.BlockSpec((H, H), lambda t, nb: (0, 0)),
            pl.BlockSpec((BLK, 1), lambda t, nb: (nb, 0)),
        ],
        out_specs=[
            pl.BlockSpec((BLK, HH), lambda t, nb: (t * NB + nb, 0)),
            pl.BlockSpec((BLK, HH), lambda t, nb: (t * NB + nb, 0)),
        ],
        out_shape=[
            jax.ShapeDtypeStruct((T * N, HH), _F32),
            jax.ShapeDtypeStruct((T * N, HH), _F32),
        ],
    )(r, st, g, be, w, u)


def _b_body(p0_ref, p1_ref, ys0_ref, ys1_ref, u_ref, bc_ref, r_ref, st_ref):
    nb = pl.program_id(1)
    p = jnp.concatenate([p0_ref[0], p1_ref[0]], axis=1)
    ysf = jnp.concatenate([ys0_ref[:], ys1_ref[:]], axis=1)
    pre = u_ref[:] * (p + ysf) + bc_ref[:][None, :]
    r = jnp.maximum(pre, 0.0)
    r_ref[:] = r
    s1 = jnp.sum(r, axis=0)
    s2 = jnp.sum(r * r, axis=0)

    @pl.when(nb == 0)
    def _():
        st_ref[0, 0] = s1
        st_ref[0, 1] = s2

    @pl.when(nb != 0)
    def _():
        st_ref[0, 0] += s1
        st_ref[0, 1] += s2


def _tc_b(p0, p1, ys0, ys1, u, bc):
    return pl.pallas_call(
        _b_body,
        grid=(T, NB),
        in_specs=[
            pl.BlockSpec((1, BLK, HH), lambda t, nb: (t, nb, 0)),
            pl.BlockSpec((1, BLK, HH), lambda t, nb: (t, nb, 0)),
            pl.BlockSpec((BLK, HH), lambda t, nb: (t * NB + nb, 0)),
            pl.BlockSpec((BLK, HH), lambda t, nb: (t * NB + nb, 0)),
            pl.BlockSpec((BLK, 1), lambda t, nb: (nb, 0)),
            pl.BlockSpec((H,), lambda t, nb: (0,)),
        ],
        out_specs=[
            pl.BlockSpec((BLK, H), lambda t, nb: (t * NB + nb, 0)),
            pl.BlockSpec((1, 8, H), lambda t, nb: (t, 0, 0)),
        ],
        out_shape=[
            jax.ShapeDtypeStruct((T * N, H), _F32),
            jax.ShapeDtypeStruct((T, 8, H), _F32),
        ],
    )(p0, p1, ys0, ys1, u, bc)


def _lstm_body(r0_ref, r1_ref, r2_ref, st_ref, g_ref, be_ref, s_ref,
               wih0_ref, whh0_ref, bi0_ref, bh0_ref,
               wih1_ref, whh1_ref, bi1_ref, bh1_ref,
               wl1h_ref, wl1s_ref, bl1_ref, wl2_ref, bl2_ref, out_ref):
    b0 = (bi0_ref[:] + bh0_ref[:])[None, :]
    b1 = (bi1_ref[:] + bh1_ref[:])[None, :]
    h0 = jnp.zeros((BLK, H), _F32)
    c0 = jnp.zeros((BLK, H), _F32)
    h1 = jnp.zeros((BLK, H), _F32)
    c1 = jnp.zeros((BLK, H), _F32)
    rrefs = (r0_ref, r1_ref, r2_ref)
    for t in range(T):
        s1 = st_ref[t, 0]
        s2 = st_ref[t, 1]
        m = s1 / N
        v = s2 / N - m * m
        sc = g_ref[:] * lax.rsqrt(v + BN_EPS)
        x = (rrefs[t][:] - m[None, :]) * sc[None, :] + be_ref[:][None, :]
        gt = (jnp.dot(x, wih0_ref[:], preferred_element_type=_F32)
              + jnp.dot(h0, whh0_ref[:], preferred_element_type=_F32) + b0)
        ig = jax.nn.sigmoid(gt[:, 0:H])
        fg = jax.nn.sigmoid(gt[:, H:2 * H])
        gg = jnp.tanh(gt[:, 2 * H:3 * H])
        og = jax.nn.sigmoid(gt[:, 3 * H:4 * H])
        c0 = fg * c0 + ig * gg
        h0 = og * jnp.tanh(c0)
        gt = (jnp.dot(h0, wih1_ref[:], preferred_element_type=_F32)
              + jnp.dot(h1, whh1_ref[:], preferred_element_type=_F32) + b1)
        ig = jax.nn.sigmoid(gt[:, 0:H])
        fg = jax.nn.sigmoid(gt[:, H:2 * H])
        gg = jnp.tanh(gt[:, 2 * H:3 * H])
        og = jax.nn.sigmoid(gt[:, 3 * H:4 * H])
        c1 = fg * c1 + ig * gg
        h1 = og * jnp.tanh(c1)
    hf = jnp.maximum(h1, 0.0)
    o = (jnp.dot(hf, wl1h_ref[:], preferred_element_type=_F32)
         + jnp.dot(s_ref[:], wl1s_ref[:], preferred_element_type=_F32)
         + bl1_ref[:][None, :])
    out_ref[:] = jnp.sum(o * wl2_ref[:], axis=1, keepdims=True) + bl2_ref[0, 0]


def _tc_lstm(r, st, g, be, s, wih0, whh0, bi0, bh0, wih1, whh1, bi1, bh1,
             wl1h, wl1s, bl1, wl2, bl2):
    vec = lambda d: pl.BlockSpec((d,), lambda nb: (0,))
    mat = lambda a, b: pl.BlockSpec((a, b), lambda nb: (0, 0))
    return pl.pallas_call(
        _lstm_body,
        grid=(NB,),
        in_specs=[
            pl.BlockSpec((BLK, H), lambda nb: (nb, 0)),
            pl.BlockSpec((BLK, H), lambda nb: (NB + nb, 0)),
            pl.BlockSpec((BLK, H), lambda nb: (2 * NB + nb, 0)),
            pl.BlockSpec((T, 8, H), lambda nb: (0, 0, 0)),
            vec(H), vec(H),
            pl.BlockSpec((BLK, 8), lambda nb: (nb, 0)),
            mat(H, 4 * H), mat(H, 4 * H), vec(4 * H), vec(4 * H),
            mat(H, 4 * H), mat(H, 4 * H), vec(4 * H), vec(4 * H),
            mat(H, H), mat(8, H), vec(H), mat(1, H), mat(1, 1),
        ],
        out_specs=pl.BlockSpec((BLK, 1), lambda nb: (nb, 0)),
        out_shape=jax.ShapeDtypeStruct((N, 1), _F32),
    )(r, r, r, st, g, be, s, wih0, whh0, bi0, bh0, wih1, whh1, bi1, bh1,
      wl1h, wl1s, bl1, wl2, bl2)


def kernel(X, edge_index, edge_weight, params):
    p = params
    src = edge_index[0]
    dst = edge_index[1]
    pad = E_PAD - E
    src_p = jnp.concatenate([src, jnp.zeros((pad,), jnp.int32)])
    dst_p = jnp.concatenate([dst, jnp.zeros((pad,), jnp.int32)])
    ew_p = jnp.concatenate([edge_weight, jnp.zeros((pad,), _F32)])
    toff = (jnp.arange(T, dtype=jnp.int32) * N)[:, None]
    src3 = (src_p[None, :] + toff).reshape(T * E_PAD // CH, CH)
    dst2 = dst_p.reshape(E_PAD // CH, CH)
    ew2 = ew_p.reshape(E_PAD // CH, CH)

    da, db = _sc_deg(dst2, ew2)
    da = da[:N].reshape(N, 1)
    db = db[:N].reshape(N, 1)

    ys0, ys1, u = _tc_a0(X, da, db, p["W0"])
    for i in range(4):
        p0, p1 = _sc_spmm(src3, dst2, ew2, ys0, ys1)
        p0 = p0.reshape(T, NA, HH)
        p1 = p1.reshape(T, NA, HH)
        r, st = _tc_b(p0, p1, ys0, ys1, u, p[f"bc{i}"])
        if i < 3:
            ys0, ys1 = _tc_ai(r, st, p[f"g{i}"], p[f"be{i}"], p[f"W{i + 1}"], u)

    s_mat = jnp.pad(X[:, :, 0].T, ((0, 0), (0, 5)))
    wl1h = p["Wl1"][:H]
    wl1s = jnp.pad(p["Wl1"][H:], ((0, 5), (0, 0)))
    out = _tc_lstm(
        r, st, p["g3"], p["be3"], s_mat,
        p["Wih0"].T, p["Whh0"].T, p["bih0"], p["bhh0"],
        p["Wih1"].T, p["Whh1"].T, p["bih1"], p["bhh1"],
        wl1h, wl1s, p["bl1"], p["Wl2"].T, p["bl2"].reshape(1, 1),
    )
    return out
